# Initial kernel scaffold; baseline (speedup 1.0000x reference)
import jax
import jax.numpy as jnp
from jax.experimental import pallas as pl


def kernel(x, edge_index, W1, b1, W3, b3, W2, b2):
    raise NotImplementedError("write your pallas kernel here")


# speedup vs baseline: 10.0356x; 10.0356x over previous
import functools

import jax
import jax.numpy as jnp
from jax import lax
from jax.experimental import pallas as pl
from jax.experimental.pallas import tpu as pltpu
from jax.experimental.pallas import tpu_sc as plsc

N = 10000
E = 320000
D_IN = 128
D_HID = 64
D_OUT = 128

NC, NS = 2, 16
NW = NC * NS
NPAD = 10240
EPAD = 327680
EPW = EPAD // NW
C = 128
NCHUNK = EPW // C
ROWS_PT = NPAD // NS

_sc_mesh = plsc.VectorSubcoreMesh(
    core_axis_name="c", subcore_axis_name="s", num_cores=NC, num_subcores=NS
)

DEG_CHUNK = 1024


@functools.partial(
    pl.kernel,
    out_type=jax.ShapeDtypeStruct((NW, NPAD), jnp.float32),
    mesh=_sc_mesh,
    scratch_types=[
        pltpu.VMEM((NPAD,), jnp.float32),
        pltpu.VMEM((DEG_CHUNK,), jnp.int32),
    ],
    compiler_params=pltpu.CompilerParams(needs_layout_passes=False),
)
def _sc_degree(dst_hbm, out_hbm, hist_v, dbuf_v):
    cid = lax.axis_index("c")
    sid = lax.axis_index("s")
    wid = sid * NC + cid

    def zero_body(j, carry):
        hist_v[pl.ds(j * 16, 16)] = jnp.zeros((16,), jnp.float32)
        return carry

    lax.fori_loop(0, NPAD // 16, zero_body, None)

    base = wid * EPW
    ones = jnp.ones((16,), jnp.float32)

    def chunk_body(i, carry):
        pltpu.sync_copy(dst_hbm.at[pl.ds(base + i * DEG_CHUNK, DEG_CHUNK)], dbuf_v)

        def inner(k, c2):
            idx = dbuf_v[pl.ds(k * 16, 16)]
            plsc.addupdate_scatter(hist_v, [idx], ones)
            return c2

        lax.fori_loop(0, DEG_CHUNK // 16, inner, None)
        return carry

    lax.fori_loop(0, EPW // DEG_CHUNK, chunk_body, None)
    pltpu.sync_copy(hist_v, out_hbm.at[wid])


@functools.partial(
    pl.kernel,
    out_type=jax.ShapeDtypeStruct((NC, NPAD, D_HID), jnp.float32),
    mesh=_sc_mesh,
    scratch_types=[
        pltpu.VMEM((C,), jnp.int32),
        pltpu.VMEM((C,), jnp.int32),
        pltpu.VMEM((C, D_HID), jnp.float32),
        pltpu.VMEM_SHARED((NPAD, D_HID), jnp.float32),
        pltpu.SemaphoreType.DMA,
    ],
    compiler_params=pltpu.CompilerParams(
        needs_layout_passes=False, use_tc_tiling_on_sc=False
    ),
)
def _sc_aggregate(y_hbm, src_hbm, dst_hbm, out_hbm, srcv, dstv, rows, acc_sh, sem):
    cid = lax.axis_index("c")
    sid = lax.axis_index("s")
    wid = sid * NC + cid

    r0 = sid * ROWS_PT
    pltpu.sync_copy(y_hbm.at[pl.ds(r0, ROWS_PT)], acc_sh.at[pl.ds(r0, ROWS_PT)])
    plsc.subcore_barrier()

    base = wid * EPW

    def chunk_body(i, carry):
        e0 = base + i * C
        pltpu.sync_copy(src_hbm.at[pl.ds(e0, C)], srcv)
        pltpu.async_copy(y_hbm.at[srcv], rows, sem).wait()
        pltpu.sync_copy(dst_hbm.at[pl.ds(e0, C)], dstv)
        pltpu.sync_copy(rows, acc_sh.at[dstv], add=True)
        return carry

    lax.fori_loop(0, NCHUNK, chunk_body, None)
    plsc.subcore_barrier()
    pltpu.sync_copy(acc_sh.at[pl.ds(r0, ROWS_PT)], out_hbm.at[cid, pl.ds(r0, ROWS_PT)])


ROW_BLK = 2048


def _tc_prep_body(hist_ref, x_ref, w_ref, y_ref, dinv_ref):
    deg = jnp.sum(hist_ref[...], axis=0) + 1.0
    dinv = lax.rsqrt(deg)
    xw = jnp.dot(x_ref[...], w_ref[...], preferred_element_type=jnp.float32)
    y_ref[...] = xw * dinv[:, None]
    dinv_ref[...] = dinv[:, None]


def _tc_prep(hist, xp, W1):
    grid = NPAD // ROW_BLK
    return pl.pallas_call(
        _tc_prep_body,
        grid=(grid,),
        in_specs=[
            pl.BlockSpec((NW, ROW_BLK), lambda i: (0, i)),
            pl.BlockSpec((ROW_BLK, D_IN), lambda i: (i, 0)),
            pl.BlockSpec((D_IN, D_HID), lambda i: (0, 0)),
        ],
        out_specs=[
            pl.BlockSpec((ROW_BLK, D_HID), lambda i: (i, 0)),
            pl.BlockSpec((ROW_BLK, 1), lambda i: (i, 0)),
        ],
        out_shape=[
            jax.ShapeDtypeStruct((NPAD, D_HID), jnp.float32),
            jax.ShapeDtypeStruct((NPAD, 1), jnp.float32),
        ],
    )(hist, xp, W1)


def _tc_mid_body(p_ref, y_ref, dinv_ref, w_ref, b_ref, out_ref):
    dinv = dinv_ref[...]
    s = p_ref[0] + p_ref[1] - y_ref[...]
    h = jnp.maximum(dinv * s + b_ref[...], 0.0)
    out_ref[...] = jnp.dot(h, w_ref[...], preferred_element_type=jnp.float32) * dinv


def _tc_mid(p, y, dinv, W, b):
    grid = NPAD // ROW_BLK
    return pl.pallas_call(
        _tc_mid_body,
        grid=(grid,),
        in_specs=[
            pl.BlockSpec((NC, ROW_BLK, D_HID), lambda i: (0, i, 0)),
            pl.BlockSpec((ROW_BLK, D_HID), lambda i: (i, 0)),
            pl.BlockSpec((ROW_BLK, 1), lambda i: (i, 0)),
            pl.BlockSpec((D_HID, D_HID), lambda i: (0, 0)),
            pl.BlockSpec((1, D_HID), lambda i: (0, 0)),
        ],
        out_specs=pl.BlockSpec((ROW_BLK, D_HID), lambda i: (i, 0)),
        out_shape=jax.ShapeDtypeStruct((NPAD, D_HID), jnp.float32),
    )(p, y, dinv, W, b)


def _tc_mid2_body(p_ref, y_ref, dinv_ref, b_ref, out_ref):
    dinv = dinv_ref[...]
    s = p_ref[0] + p_ref[1] - y_ref[...]
    out_ref[...] = dinv * jnp.maximum(dinv * s + b_ref[...], 0.0)


def _tc_mid2(p, y, dinv, b):
    grid = NPAD // ROW_BLK
    return pl.pallas_call(
        _tc_mid2_body,
        grid=(grid,),
        in_specs=[
            pl.BlockSpec((NC, ROW_BLK, D_HID), lambda i: (0, i, 0)),
            pl.BlockSpec((ROW_BLK, D_HID), lambda i: (i, 0)),
            pl.BlockSpec((ROW_BLK, 1), lambda i: (i, 0)),
            pl.BlockSpec((1, D_HID), lambda i: (0, 0)),
        ],
        out_specs=pl.BlockSpec((ROW_BLK, D_HID), lambda i: (i, 0)),
        out_shape=jax.ShapeDtypeStruct((NPAD, D_HID), jnp.float32),
    )(p, y, dinv, b)


def _tc_fin_body(p_ref, y_ref, dinv_ref, w_ref, b_ref, out_ref):
    dinv = dinv_ref[...]
    z = dinv * (p_ref[0] + p_ref[1] - y_ref[...])
    zw = jnp.dot(z, w_ref[...], preferred_element_type=jnp.float32)
    out_ref[...] = jnp.maximum(zw + b_ref[...], 0.0)


def _tc_fin(p, y, dinv, W, b):
    grid = NPAD // ROW_BLK
    return pl.pallas_call(
        _tc_fin_body,
        grid=(grid,),
        in_specs=[
            pl.BlockSpec((NC, ROW_BLK, D_HID), lambda i: (0, i, 0)),
            pl.BlockSpec((ROW_BLK, D_HID), lambda i: (i, 0)),
            pl.BlockSpec((ROW_BLK, 1), lambda i: (i, 0)),
            pl.BlockSpec((D_HID, D_OUT), lambda i: (0, 0)),
            pl.BlockSpec((1, D_OUT), lambda i: (0, 0)),
        ],
        out_specs=pl.BlockSpec((ROW_BLK, D_OUT), lambda i: (i, 0)),
        out_shape=jax.ShapeDtypeStruct((NPAD, D_OUT), jnp.float32),
    )(p, y, dinv, W, b)


def kernel(x, edge_index, W1, b1, W3, b3, W2, b2):
    pad_idx = jnp.full((EPAD - E,), N, dtype=jnp.int32)
    srcp = jnp.concatenate([edge_index[0], pad_idx])
    dstp = jnp.concatenate([edge_index[1], pad_idx])
    xp = jnp.pad(x, ((0, NPAD - N), (0, 0)))

    hist = _sc_degree(dstp)
    y1, dinv = _tc_prep(hist, xp, W1)
    p1 = _sc_aggregate(y1, srcp, dstp)
    y2 = _tc_mid(p1, y1, dinv, W3, b1.reshape(1, -1))
    p2 = _sc_aggregate(y2, srcp, dstp)
    y3 = _tc_mid2(p2, y2, dinv, b3.reshape(1, -1))
    p3 = _sc_aggregate(y3, srcp, dstp)
    out = _tc_fin(p3, y3, dinv, W2, b2.reshape(1, -1))
    return out[:N]


# speedup vs baseline: 14.8719x; 1.4819x over previous
import functools

import jax
import jax.numpy as jnp
from jax import lax
from jax.experimental import pallas as pl
from jax.experimental.pallas import tpu as pltpu
from jax.experimental.pallas import tpu_sc as plsc

N = 10000
E = 320000
D_IN = 128
D_HID = 64
D_OUT = 128

NC, NS = 2, 16
NW = NC * NS
NPAD = 10240
EPAD = 327680
EPW = EPAD // NW
C = 128
NCHUNK = EPW // C
ROWS_PT = NPAD // NS

_sc_mesh = plsc.VectorSubcoreMesh(
    core_axis_name="c", subcore_axis_name="s", num_cores=NC, num_subcores=NS
)

DEG_CHUNK = 1024


@functools.partial(
    pl.kernel,
    out_type=jax.ShapeDtypeStruct((NW, NPAD), jnp.float32),
    mesh=_sc_mesh,
    scratch_types=[
        pltpu.VMEM((NPAD,), jnp.float32),
        pltpu.VMEM((DEG_CHUNK,), jnp.int32),
    ],
    compiler_params=pltpu.CompilerParams(needs_layout_passes=False),
)
def _sc_degree(dst_hbm, out_hbm, hist_v, dbuf_v):
    cid = lax.axis_index("c")
    sid = lax.axis_index("s")
    wid = sid * NC + cid

    def zero_body(j, carry):
        hist_v[pl.ds(j * 16, 16)] = jnp.zeros((16,), jnp.float32)
        return carry

    lax.fori_loop(0, NPAD // 16, zero_body, None)

    base = wid * EPW
    ones = jnp.ones((16,), jnp.float32)

    def chunk_body(i, carry):
        pltpu.sync_copy(dst_hbm.at[pl.ds(base + i * DEG_CHUNK, DEG_CHUNK)], dbuf_v)

        def inner(k, c2):
            idx = dbuf_v[pl.ds(k * 16, 16)]
            plsc.addupdate_scatter(hist_v, [idx], ones)
            return c2

        lax.fori_loop(0, DEG_CHUNK // 16, inner, None)
        return carry

    lax.fori_loop(0, EPW // DEG_CHUNK, chunk_body, None)
    pltpu.sync_copy(hist_v, out_hbm.at[wid])


K = 4
NG = NCHUNK // K


@functools.partial(
    pl.kernel,
    out_type=jax.ShapeDtypeStruct((NC, NPAD, D_HID), jnp.float32),
    mesh=_sc_mesh,
    scratch_types=[
        pltpu.VMEM((NCHUNK, C), jnp.int32),
        pltpu.VMEM((NCHUNK, C), jnp.int32),
        pltpu.VMEM((2 * K, C, D_HID), jnp.float32),
        pltpu.VMEM_SHARED((NPAD, D_HID), jnp.float32),
        pltpu.SemaphoreType.DMA,
        pltpu.SemaphoreType.DMA,
    ],
    compiler_params=pltpu.CompilerParams(
        needs_layout_passes=False, use_tc_tiling_on_sc=False
    ),
)
def _sc_aggregate(y_hbm, src_hbm, dst_hbm, out_hbm, srci, dsti, rows, acc_sh,
                  sem_a, sem_b):
    cid = lax.axis_index("c")
    sid = lax.axis_index("s")
    wid = sid * NC + cid

    pltpu.sync_copy(src_hbm.at[wid], srci)
    pltpu.sync_copy(dst_hbm.at[wid], dsti)

    r0 = sid * ROWS_PT
    pltpu.sync_copy(y_hbm.at[pl.ds(r0, ROWS_PT)], acc_sh.at[pl.ds(r0, ROWS_PT)])
    plsc.subcore_barrier()

    def fire(g, bufset, sem):
        for b in range(K):
            pltpu.async_copy(y_hbm.at[srci.at[g * K + b]],
                             rows.at[bufset * K + b], sem)

    def drain_scatter(g, bufset, sem):
        for b in range(K):
            pltpu.make_async_copy(y_hbm.at[pl.ds(0, C)],
                                  rows.at[bufset * K + b], sem).wait()
        for b in range(K):
            pltpu.sync_copy(rows.at[bufset * K + b],
                            acc_sh.at[dsti.at[g * K + b]], add=True)

    fire(0, 0, sem_a)
    fire(1, 1, sem_b)

    def body(t, carry):
        g = 2 * t
        drain_scatter(g, 0, sem_a)
        fire(g + 2, 0, sem_a)
        drain_scatter(g + 1, 1, sem_b)
        fire(g + 3, 1, sem_b)
        return carry

    lax.fori_loop(0, NG // 2 - 1, body, None)
    drain_scatter(NG - 2, 0, sem_a)
    drain_scatter(NG - 1, 1, sem_b)

    plsc.subcore_barrier()
    pltpu.sync_copy(acc_sh.at[pl.ds(r0, ROWS_PT)], out_hbm.at[cid, pl.ds(r0, ROWS_PT)])


ROW_BLK = 2048


def _tc_prep_body(hist_ref, x_ref, w_ref, y_ref, dinv_ref):
    deg = jnp.sum(hist_ref[...], axis=0) + 1.0
    dinv = lax.rsqrt(deg)
    xw = jnp.dot(x_ref[...], w_ref[...], preferred_element_type=jnp.float32)
    y_ref[...] = xw * dinv[:, None]
    dinv_ref[...] = dinv[:, None]


def _tc_prep(hist, xp, W1):
    grid = NPAD // ROW_BLK
    return pl.pallas_call(
        _tc_prep_body,
        grid=(grid,),
        in_specs=[
            pl.BlockSpec((NW, ROW_BLK), lambda i: (0, i)),
            pl.BlockSpec((ROW_BLK, D_IN), lambda i: (i, 0)),
            pl.BlockSpec((D_IN, D_HID), lambda i: (0, 0)),
        ],
        out_specs=[
            pl.BlockSpec((ROW_BLK, D_HID), lambda i: (i, 0)),
            pl.BlockSpec((ROW_BLK, 1), lambda i: (i, 0)),
        ],
        out_shape=[
            jax.ShapeDtypeStruct((NPAD, D_HID), jnp.float32),
            jax.ShapeDtypeStruct((NPAD, 1), jnp.float32),
        ],
    )(hist, xp, W1)


def _tc_mid_body(p_ref, y_ref, dinv_ref, w_ref, b_ref, out_ref):
    dinv = dinv_ref[...]
    s = p_ref[0] + p_ref[1] - y_ref[...]
    h = jnp.maximum(dinv * s + b_ref[...], 0.0)
    out_ref[...] = jnp.dot(h, w_ref[...], preferred_element_type=jnp.float32) * dinv


def _tc_mid(p, y, dinv, W, b):
    grid = NPAD // ROW_BLK
    return pl.pallas_call(
        _tc_mid_body,
        grid=(grid,),
        in_specs=[
            pl.BlockSpec((NC, ROW_BLK, D_HID), lambda i: (0, i, 0)),
            pl.BlockSpec((ROW_BLK, D_HID), lambda i: (i, 0)),
            pl.BlockSpec((ROW_BLK, 1), lambda i: (i, 0)),
            pl.BlockSpec((D_HID, D_HID), lambda i: (0, 0)),
            pl.BlockSpec((1, D_HID), lambda i: (0, 0)),
        ],
        out_specs=pl.BlockSpec((ROW_BLK, D_HID), lambda i: (i, 0)),
        out_shape=jax.ShapeDtypeStruct((NPAD, D_HID), jnp.float32),
    )(p, y, dinv, W, b)


def _tc_mid2_body(p_ref, y_ref, dinv_ref, b_ref, out_ref):
    dinv = dinv_ref[...]
    s = p_ref[0] + p_ref[1] - y_ref[...]
    out_ref[...] = dinv * jnp.maximum(dinv * s + b_ref[...], 0.0)


def _tc_mid2(p, y, dinv, b):
    grid = NPAD // ROW_BLK
    return pl.pallas_call(
        _tc_mid2_body,
        grid=(grid,),
        in_specs=[
            pl.BlockSpec((NC, ROW_BLK, D_HID), lambda i: (0, i, 0)),
            pl.BlockSpec((ROW_BLK, D_HID), lambda i: (i, 0)),
            pl.BlockSpec((ROW_BLK, 1), lambda i: (i, 0)),
            pl.BlockSpec((1, D_HID), lambda i: (0, 0)),
        ],
        out_specs=pl.BlockSpec((ROW_BLK, D_HID), lambda i: (i, 0)),
        out_shape=jax.ShapeDtypeStruct((NPAD, D_HID), jnp.float32),
    )(p, y, dinv, b)


def _tc_fin_body(p_ref, y_ref, dinv_ref, w_ref, b_ref, out_ref):
    dinv = dinv_ref[...]
    z = dinv * (p_ref[0] + p_ref[1] - y_ref[...])
    zw = jnp.dot(z, w_ref[...], preferred_element_type=jnp.float32)
    out_ref[...] = jnp.maximum(zw + b_ref[...], 0.0)


def _tc_fin(p, y, dinv, W, b):
    grid = NPAD // ROW_BLK
    return pl.pallas_call(
        _tc_fin_body,
        grid=(grid,),
        in_specs=[
            pl.BlockSpec((NC, ROW_BLK, D_HID), lambda i: (0, i, 0)),
            pl.BlockSpec((ROW_BLK, D_HID), lambda i: (i, 0)),
            pl.BlockSpec((ROW_BLK, 1), lambda i: (i, 0)),
            pl.BlockSpec((D_HID, D_OUT), lambda i: (0, 0)),
            pl.BlockSpec((1, D_OUT), lambda i: (0, 0)),
        ],
        out_specs=pl.BlockSpec((ROW_BLK, D_OUT), lambda i: (i, 0)),
        out_shape=jax.ShapeDtypeStruct((NPAD, D_OUT), jnp.float32),
    )(p, y, dinv, W, b)


def kernel(x, edge_index, W1, b1, W3, b3, W2, b2):
    pad_idx = jnp.full((EPAD - E,), N, dtype=jnp.int32)
    srcp = jnp.concatenate([edge_index[0], pad_idx])
    dstp = jnp.concatenate([edge_index[1], pad_idx])
    xp = jnp.pad(x, ((0, NPAD - N), (0, 0)))

    src3 = srcp.reshape(NW, NCHUNK, C)
    dst3 = dstp.reshape(NW, NCHUNK, C)

    hist = _sc_degree(dstp)
    y1, dinv = _tc_prep(hist, xp, W1)
    p1 = _sc_aggregate(y1, src3, dst3)
    y2 = _tc_mid(p1, y1, dinv, W3, b1.reshape(1, -1))
    p2 = _sc_aggregate(y2, src3, dst3)
    y3 = _tc_mid2(p2, y2, dinv, b3.reshape(1, -1))
    p3 = _sc_aggregate(y3, src3, dst3)
    out = _tc_fin(p3, y3, dinv, W2, b2.reshape(1, -1))
    return out[:N]
